# Initial kernel scaffold; baseline (speedup 1.0000x reference)
import jax
import jax.numpy as jnp
from jax.experimental import pallas as pl


def kernel(x, indices, w1, bias1, gamma1, beta1, mean1, var1, w2, bias2, gamma2, beta2, mean2, var2):
    raise NotImplementedError("write your pallas kernel here")


# speedup vs baseline: 1.7651x; 1.7651x over previous
import jax
import jax.numpy as jnp
from jax import lax
from jax.experimental import pallas as pl
from jax.experimental.pallas import tpu as pltpu

_BF = jnp.bfloat16
_F32 = jnp.float32


def _make_body(H, W, Cin, Cout, CM):
    W2, H2 = 2 * W, 2 * H
    M2 = H2 * W2
    NC = M2 // CM
    OFFU = W2 + 16
    BASEH = W2 + 16

    def body(xc_ref, s4_ref, w1_ref, s1_ref, b1_ref, w2_ref, s2_ref, b2_ref,
             out_ref, u3, h3):
        ri = lax.broadcasted_iota(jnp.int32, (W2, 1), 0)
        par = (ri % 2).astype(_F32)
        m_dn = ri < (W2 - 1)
        m_up = ri > 0
        ci = lax.broadcasted_iota(jnp.int32, (CM, 1), 0)

        zb_u = jnp.zeros((W2, 3 * Cin), _BF)
        zs_u = jnp.zeros((8, 3 * Cin), _BF)
        zb_h = jnp.zeros((W2, 3 * Cout), _BF)
        zs_h = jnp.zeros((8, 3 * Cout), _BF)

        u3[pl.ds(OFFU - W2, W2), :] = zb_u
        u3[pl.ds(OFFU + M2, W2), :] = zb_u
        u3[pl.ds(OFFU, 8), :] = zs_u
        u3[pl.ds(OFFU + M2 - 8, 8), :] = zs_u
        h3[pl.ds(BASEH - W2, W2), :] = zb_h
        h3[pl.ds(BASEH + M2, W2), :] = zb_h
        h3[pl.ds(BASEH, 8), :] = zs_h
        h3[pl.ds(BASEH + M2 - 8, 8), :] = zs_h

        S4 = s4_ref[...]
        for g in range(H // 2):
            pair = xc_ref[0, pl.ds(g * 4 * W, 4 * W), :]
            E = jnp.dot(S4, pair, preferred_element_type=_F32)
            for t in range(2):
                xs = E[2 * W2 * t: 2 * W2 * t + W2]
                cs = E[2 * W2 * t + W2: 2 * W2 * (t + 1)]
                for a in range(2):
                    keep = cs == (par + float(2 * a))
                    urow = jnp.where(keep, xs, 0.0).astype(_BF)
                    r0 = OFFU + (2 * (2 * g + t) + a) * W2
                    u3[pl.ds(r0, W2), Cin:2 * Cin] = urow
                    u3[pl.ds(r0 + 1, W2), 0:Cin] = jnp.where(m_dn, urow, 0)
                    u3[pl.ds(r0 - 1, W2), 2 * Cin:3 * Cin] = jnp.where(
                        m_up, urow, 0)

        s1 = s1_ref[...]
        b1 = b1_ref[...]
        for c in range(NC):
            o0 = c * CM
            acc = None
            for dyi in range(3):
                blk = u3[pl.ds(OFFU + o0 + (dyi - 1) * W2, CM), :]
                d = jnp.dot(blk, w1_ref[dyi], preferred_element_type=_F32)
                acc = d if acc is None else acc + d
            y = jnp.maximum(acc * s1 + b1, 0.0).astype(_BF)
            mk0 = ((ci + (o0 + 1 + W2)) % W2) != 0
            mk2 = ((ci + (o0 - 1 + W2)) % W2) != (W2 - 1)
            h3[pl.ds(BASEH + o0, CM), Cout:2 * Cout] = y
            h3[pl.ds(BASEH + o0 + 1, CM), 0:Cout] = jnp.where(mk0, y, 0)
            h3[pl.ds(BASEH + o0 - 1, CM), 2 * Cout:3 * Cout] = jnp.where(
                mk2, y, 0)

        s2 = s2_ref[...]
        b2 = b2_ref[...]
        for c in range(NC):
            o0 = c * CM
            acc = None
            for dyi in range(3):
                blk = h3[pl.ds(BASEH + o0 + (dyi - 1) * W2, CM), :]
                d = jnp.dot(blk, w2_ref[dyi], preferred_element_type=_F32)
                acc = d if acc is None else acc + d
            y = jnp.maximum(acc * s2 + b2, 0.0)
            out_ref[0, pl.ds(o0, CM), :] = y.astype(out_ref.dtype)

    return body


def kernel(x, indices, w1, bias1, gamma1, beta1, mean1, var1,
           w2, bias2, gamma2, beta2, mean2, var2, *, interpret=False):
    N, Cin, H, W = x.shape
    Cout = w1.shape[0]
    W2, H2 = 2 * W, 2 * H
    M2 = H2 * W2
    assert H % 2 == 0 and W % 4 == 0
    eps = 1e-5

    def fold(gamma, beta, mean, var, cbias):
        s = gamma / jnp.sqrt(var + eps)
        b = (cbias - mean) * s + beta
        return s.reshape(1, -1).astype(_F32), b.reshape(1, -1).astype(_F32)

    s1, b1 = fold(gamma1, beta1, mean1, var1, bias1)
    s2, b2 = fold(gamma2, beta2, mean2, var2, bias2)

    def pack(w):
        t = jnp.transpose(w, (2, 3, 1, 0))
        return t.reshape(3, 3 * w.shape[1], w.shape[0]).astype(_BF)

    w1p, w2p = pack(w1), pack(w2)

    x_cl = jnp.transpose(x, (0, 2, 3, 1)).astype(_BF)
    code = (2 * ((indices // W2) % 2) + (indices % 2)).astype(_BF)
    code_cl = jnp.transpose(code, (0, 2, 3, 1))
    xc = jnp.concatenate([x_cl, code_cl], axis=2)
    xc = xc.reshape(N, H * 2 * W, Cin)

    w2i = jnp.arange(W2)
    top = (w2i[:, None] // 2 == jnp.arange(2 * W)[None, :]).astype(_BF)
    bot = (W + w2i[:, None] // 2 == jnp.arange(2 * W)[None, :]).astype(_BF)
    s2h = jnp.concatenate([top, bot], axis=0)
    z = jnp.zeros_like(s2h)
    s4 = jnp.concatenate(
        [jnp.concatenate([s2h, z], axis=1),
         jnp.concatenate([z, s2h], axis=1)], axis=0)

    CM = 1024 if M2 % 1024 == 0 else W2
    body = _make_body(H, W, Cin, Cout, CM)
    OFF = W2 + 16
    u3_rows = OFF + M2 + W2
    h3_rows = OFF + M2 + W2

    out_flat = pl.pallas_call(
        body,
        out_shape=jax.ShapeDtypeStruct((N, M2, Cout), _F32),
        grid=(N,),
        in_specs=[
            pl.BlockSpec((1, H * 2 * W, Cin), lambda n: (n, 0, 0)),
            pl.BlockSpec((4 * W2, 4 * W), lambda n: (0, 0)),
            pl.BlockSpec((3, 3 * Cin, Cout), lambda n: (0, 0, 0)),
            pl.BlockSpec((1, Cout), lambda n: (0, 0)),
            pl.BlockSpec((1, Cout), lambda n: (0, 0)),
            pl.BlockSpec((3, 3 * Cout, Cout), lambda n: (0, 0, 0)),
            pl.BlockSpec((1, Cout), lambda n: (0, 0)),
            pl.BlockSpec((1, Cout), lambda n: (0, 0)),
        ],
        out_specs=pl.BlockSpec((1, M2, Cout), lambda n: (n, 0, 0)),
        scratch_shapes=[
            pltpu.VMEM((u3_rows, 3 * Cin), _BF),
            pltpu.VMEM((h3_rows, 3 * Cout), _BF),
        ],
        compiler_params=pltpu.CompilerParams(
            dimension_semantics=("parallel",),
            vmem_limit_bytes=48 * 1024 * 1024),
        interpret=interpret,
    )(xc, s4, w1p, s1, b1, w2p, s2, b2)

    out = out_flat.reshape(N, H2, W2, Cout)
    return jnp.transpose(out, (0, 3, 1, 2))


# speedup vs baseline: 1.8695x; 1.0591x over previous
import jax
import jax.numpy as jnp
from jax import lax
from jax.experimental import pallas as pl
from jax.experimental.pallas import tpu as pltpu

_BF = jnp.bfloat16
_F32 = jnp.float32


def _make_body(H, W, Cin, Cout, CM):
    W2, H2 = 2 * W, 2 * H
    M2 = H2 * W2
    NC = M2 // CM
    OFFU = W2 + 16
    BASEH = W2 + 16

    def body(x_ref, ind_ref, s4_ref, w1_ref, s1_ref, b1_ref,
             w2_ref, s2_ref, b2_ref, out_ref, u3, h3):
        ri = lax.broadcasted_iota(jnp.int32, (W2, 1), 0)
        par = (ri % 2).astype(_F32)
        m_dn = ri < (W2 - 1)
        m_up = ri > 0
        ci = lax.broadcasted_iota(jnp.int32, (CM, 1), 0)

        zb_u = jnp.zeros((W2, 3 * Cin), _BF)
        zs_u = jnp.zeros((8, 3 * Cin), _BF)
        zb_h = jnp.zeros((W2, 3 * Cout), _BF)
        zs_h = jnp.zeros((8, 3 * Cout), _BF)

        u3[pl.ds(OFFU - W2, W2), :] = zb_u
        u3[pl.ds(OFFU + M2, W2), :] = zb_u
        u3[pl.ds(OFFU, 8), :] = zs_u
        u3[pl.ds(OFFU + M2 - 8, 8), :] = zs_u
        h3[pl.ds(BASEH - W2, W2), :] = zb_h
        h3[pl.ds(BASEH + M2, W2), :] = zb_h
        h3[pl.ds(BASEH, 8), :] = zs_h
        h3[pl.ds(BASEH + M2 - 8, 8), :] = zs_h

        S4 = s4_ref[...]
        for g in range(H // 2):
            xch = x_ref[0, :, pl.ds(g * 2 * W, 2 * W)].astype(_BF)
            ich = ind_ref[0, :, pl.ds(g * 2 * W, 2 * W)]
            cch = (2 * ((ich // W2) % 2) + (ich % 2)).astype(_BF)
            pair = jnp.concatenate(
                [jnp.transpose(xch), jnp.transpose(cch)], axis=0)
            E = jnp.dot(S4, pair, preferred_element_type=_F32)
            for t in range(2):
                xs = E[2 * W2 * t: 2 * W2 * t + W2]
                cs = E[2 * W2 * t + W2: 2 * W2 * (t + 1)]
                for a in range(2):
                    keep = cs == (par + float(2 * a))
                    urow = jnp.where(keep, xs, 0.0).astype(_BF)
                    r0 = OFFU + (2 * (2 * g + t) + a) * W2
                    u3[pl.ds(r0, W2), Cin:2 * Cin] = urow
                    u3[pl.ds(r0 + 1, W2), 0:Cin] = jnp.where(m_dn, urow, 0)
                    u3[pl.ds(r0 - 1, W2), 2 * Cin:3 * Cin] = jnp.where(
                        m_up, urow, 0)

        s1 = s1_ref[...]
        b1 = b1_ref[...]
        for c in range(NC):
            o0 = c * CM
            acc = None
            for dyi in range(3):
                blk = u3[pl.ds(OFFU + o0 + (dyi - 1) * W2, CM), :]
                d = jnp.dot(blk, w1_ref[dyi], preferred_element_type=_F32)
                acc = d if acc is None else acc + d
            y = jnp.maximum(acc * s1 + b1, 0.0).astype(_BF)
            mk0 = ((ci + (o0 + 1 + W2)) % W2) != 0
            mk2 = ((ci + (o0 - 1 + W2)) % W2) != (W2 - 1)
            h3[pl.ds(BASEH + o0, CM), Cout:2 * Cout] = y
            h3[pl.ds(BASEH + o0 + 1, CM), 0:Cout] = jnp.where(mk0, y, 0)
            h3[pl.ds(BASEH + o0 - 1, CM), 2 * Cout:3 * Cout] = jnp.where(
                mk2, y, 0)

        s2 = s2_ref[...]
        b2 = b2_ref[...]
        for c in range(NC):
            o0 = c * CM
            acc = None
            for dyi in range(3):
                blk = h3[pl.ds(BASEH + o0 + (dyi - 1) * W2, CM), :]
                d = jnp.dot(blk, w2_ref[dyi], preferred_element_type=_F32)
                acc = d if acc is None else acc + d
            y = jnp.maximum(acc * s2 + b2, 0.0)
            out_ref[0, :, pl.ds(o0, CM)] = jnp.transpose(y).astype(
                out_ref.dtype)

    return body


def kernel(x, indices, w1, bias1, gamma1, beta1, mean1, var1,
           w2, bias2, gamma2, beta2, mean2, var2, *, interpret=False):
    N, Cin, H, W = x.shape
    Cout = w1.shape[0]
    W2, H2 = 2 * W, 2 * H
    M2 = H2 * W2
    assert H % 2 == 0 and W % 4 == 0
    eps = 1e-5

    def fold(gamma, beta, mean, var, cbias):
        s = gamma / jnp.sqrt(var + eps)
        b = (cbias - mean) * s + beta
        return s.reshape(1, -1).astype(_F32), b.reshape(1, -1).astype(_F32)

    s1, b1 = fold(gamma1, beta1, mean1, var1, bias1)
    s2, b2 = fold(gamma2, beta2, mean2, var2, bias2)

    def pack(w):
        t = jnp.transpose(w, (2, 3, 1, 0))
        return t.reshape(3, 3 * w.shape[1], w.shape[0]).astype(_BF)

    w1p, w2p = pack(w1), pack(w2)

    w2i = jnp.arange(W2)[:, None]
    cols = jnp.arange(4 * W)[None, :]
    blk = [(w2i // 2 == cols - off).astype(_BF) for off in (0, W, 2 * W, 3 * W)]
    s4 = jnp.concatenate(
        [blk[0], blk[2], blk[1], blk[3]], axis=0)

    CM = 1024 if M2 % 1024 == 0 else W2
    body = _make_body(H, W, Cin, Cout, CM)
    OFF = W2 + 16
    u3_rows = OFF + M2 + W2
    h3_rows = OFF + M2 + W2

    out_flat = pl.pallas_call(
        body,
        out_shape=jax.ShapeDtypeStruct((N, Cout, M2), _F32),
        grid=(N,),
        in_specs=[
            pl.BlockSpec((1, Cin, H * W), lambda n: (n, 0, 0)),
            pl.BlockSpec((1, Cin, H * W), lambda n: (n, 0, 0)),
            pl.BlockSpec((4 * W2, 4 * W), lambda n: (0, 0)),
            pl.BlockSpec((3, 3 * Cin, Cout), lambda n: (0, 0, 0)),
            pl.BlockSpec((1, Cout), lambda n: (0, 0)),
            pl.BlockSpec((1, Cout), lambda n: (0, 0)),
            pl.BlockSpec((3, 3 * Cout, Cout), lambda n: (0, 0, 0)),
            pl.BlockSpec((1, Cout), lambda n: (0, 0)),
            pl.BlockSpec((1, Cout), lambda n: (0, 0)),
        ],
        out_specs=pl.BlockSpec((1, Cout, M2), lambda n: (n, 0, 0)),
        scratch_shapes=[
            pltpu.VMEM((u3_rows, 3 * Cin), _BF),
            pltpu.VMEM((h3_rows, 3 * Cout), _BF),
        ],
        compiler_params=pltpu.CompilerParams(
            dimension_semantics=("parallel",),
            vmem_limit_bytes=48 * 1024 * 1024),
        interpret=interpret,
    )(x.reshape(N, Cin, H * W), indices.reshape(N, Cin, H * W),
      s4, w1p, s1, b1, w2p, s2, b2)

    return out_flat.reshape(N, Cout, H2, W2)
